# baseline (speedup 1.0000x reference)
import jax
import jax.numpy as jnp
from jax.experimental import pallas as pl
from jax.experimental.pallas import tpu as pltpu

B, L, E, FEAT, D_IN, D_H, D_OUT = 16384, 8, 8, 32, 64, 128, 32


def _fused_kernel(feat_ref, p_ref, w0_ref, w1_ref, ids_ref, out_ref):
    bb = feat_ref.shape[0]
    p = p_ref[...]
    feats = feat_ref[...]
    ids = ids_ref[...]
    w0 = w0_ref[...]
    w1 = w1_ref[...]

    iota8 = jax.lax.broadcasted_iota(jnp.int32, (bb, E), 1)

    for n in range(L):
        idn = ids[:, n:n + 1]
        oh = (iota8 == idn).astype(jnp.bfloat16)
        xb = jnp.concatenate(
            [p.astype(jnp.bfloat16), feats[:, n * FEAT:(n + 1) * FEAT]],
            axis=1)
        xm = jnp.concatenate(
            [jnp.where(idn == i, xb, jnp.bfloat16(0.0)) for i in range(E)]
            + [oh], axis=1)
        pre = jnp.dot(xm, w0, preferred_element_type=jnp.float32)
        h = jnp.tanh(pre).astype(jnp.bfloat16)
        hb = jnp.concatenate([h, oh], axis=1)
        o8 = jnp.dot(hb, w1, preferred_element_type=jnp.float32)
        o = o8[:, 0:D_OUT]
        for i in range(1, E):
            o = jnp.where(idn == i, o8[:, i * D_OUT:(i + 1) * D_OUT], o)
        p = o
    out_ref[...] = jnp.maximum(p, 0.0)


def kernel(mod_feat_seq, p_in, W0, b0, W1, b1, mod_id_seq):
    w0stack = jnp.transpose(W0, (0, 2, 1)).reshape(E * D_IN, D_H)
    w0full = jnp.concatenate([w0stack, b0], axis=0).astype(jnp.bfloat16)
    w1all = jnp.transpose(W1, (2, 0, 1)).reshape(D_H, E * D_OUT)
    b1block = jnp.einsum("ai,ij->aij", jnp.eye(E, dtype=b1.dtype),
                         b1).reshape(E, E * D_OUT)
    w1full = jnp.concatenate([w1all, b1block], axis=0).astype(jnp.bfloat16)

    feats = mod_feat_seq.reshape(B, L * FEAT).astype(jnp.bfloat16)
    ids = mod_id_seq.astype(jnp.int32)

    BB = 1024
    grid = (B // BB,)
    return pl.pallas_call(
        _fused_kernel,
        grid=grid,
        in_specs=[
            pl.BlockSpec((BB, L * FEAT), lambda b: (b, 0)),
            pl.BlockSpec((BB, D_OUT), lambda b: (b, 0)),
            pl.BlockSpec((E * D_IN + E, D_H), lambda b: (0, 0)),
            pl.BlockSpec((D_H + E, E * D_OUT), lambda b: (0, 0)),
            pl.BlockSpec((BB, L), lambda b: (b, 0)),
        ],
        out_specs=pl.BlockSpec((BB, D_OUT), lambda b: (b, 0)),
        out_shape=jax.ShapeDtypeStruct((B, D_OUT), jnp.float32),
        compiler_params=pltpu.CompilerParams(
            dimension_semantics=("parallel",)),
    )(feats, p_in, w0full, w1full, ids)


# speedup vs baseline: 1.0156x; 1.0156x over previous
import jax
import jax.numpy as jnp
from jax.experimental import pallas as pl
from jax.experimental.pallas import tpu as pltpu

B, L, E, FEAT, D_IN, D_H, D_OUT = 16384, 8, 8, 32, 64, 128, 32
FE = FEAT + 2


def _fused_kernel(feat_ref, p_ref, w0_ref, w1_ref, b1_ref, ids_ref, out_ref):
    bb = feat_ref.shape[0]
    p = p_ref[...]
    feats = feat_ref[...]
    ids = ids_ref[...]
    w0 = w0_ref[...]
    w1 = w1_ref[...]
    b1 = b1_ref[...]

    for n in range(L):
        idn = ids[:, n:n + 1]
        x = jnp.concatenate(
            [p.astype(jnp.bfloat16), feats[:, n * FE:(n + 1) * FE]],
            axis=1)
        pre = jnp.dot(x, w0,
                      preferred_element_type=jnp.float32).astype(jnp.bfloat16)
        psel = pre[:, 0:D_H]
        for i in range(1, E):
            psel = jnp.where(idn == i, pre[:, i * D_H:(i + 1) * D_H], psel)
        h = jnp.tanh(psel.astype(jnp.float32)).astype(jnp.bfloat16)
        o8 = jnp.dot(h, w1, preferred_element_type=jnp.float32)
        o = o8[:, 0:D_OUT]
        bsel = b1[0:1]
        for i in range(1, E):
            o = jnp.where(idn == i, o8[:, i * D_OUT:(i + 1) * D_OUT], o)
            bsel = jnp.where(idn == i, b1[i:i + 1], bsel)
        p = o + bsel
    out_ref[...] = jnp.maximum(p, 0.0)


def kernel(mod_feat_seq, p_in, W0, b0, W1, b1, mod_id_seq):
    w0cat = jnp.transpose(W0, (2, 0, 1)).reshape(D_IN, E * D_H)
    w0full = jnp.concatenate(
        [w0cat, b0.reshape(1, E * D_H),
         jnp.zeros((1, E * D_H), b0.dtype)], axis=0).astype(jnp.bfloat16)
    w1all = jnp.transpose(W1, (2, 0, 1)).reshape(D_H, E * D_OUT)
    w1all = w1all.astype(jnp.bfloat16)

    ones = jnp.ones((B, L, 1), mod_feat_seq.dtype)
    zeros = jnp.zeros((B, L, 1), mod_feat_seq.dtype)
    feats = jnp.concatenate([mod_feat_seq, ones, zeros], axis=2)
    feats = feats.reshape(B, L * FE).astype(jnp.bfloat16)
    ids = mod_id_seq.astype(jnp.int32)

    BB = 1024
    grid = (B // BB,)
    return pl.pallas_call(
        _fused_kernel,
        grid=grid,
        in_specs=[
            pl.BlockSpec((BB, L * FE), lambda b: (b, 0)),
            pl.BlockSpec((BB, D_OUT), lambda b: (b, 0)),
            pl.BlockSpec((D_IN + 2, E * D_H), lambda b: (0, 0)),
            pl.BlockSpec((D_H, E * D_OUT), lambda b: (0, 0)),
            pl.BlockSpec((E, D_OUT), lambda b: (0, 0)),
            pl.BlockSpec((BB, L), lambda b: (b, 0)),
        ],
        out_specs=pl.BlockSpec((BB, D_OUT), lambda b: (b, 0)),
        out_shape=jax.ShapeDtypeStruct((B, D_OUT), jnp.float32),
        compiler_params=pltpu.CompilerParams(
            dimension_semantics=("parallel",)),
    )(feats, p_in, w0full, w1all, b1, ids)


# speedup vs baseline: 1.0551x; 1.0389x over previous
import jax
import jax.numpy as jnp
from jax.experimental import pallas as pl
from jax.experimental.pallas import tpu as pltpu

B, L, E, FEAT, D_IN, D_H, D_OUT = 16384, 8, 8, 32, 64, 128, 32
FE = FEAT + 8
XW = D_OUT + FE


def _fused_kernel(feat_ref, p_ref, w0_ref, w1_ref, b1_ref, ids_ref, out_ref):
    bb = feat_ref.shape[0]
    p = p_ref[...]
    feats = feat_ref[...]
    ids = ids_ref[...]
    w0 = w0_ref[...]
    w1 = w1_ref[...]
    b1 = b1_ref[...]

    for n in range(L):
        idn = ids[:, n:n + 1]
        x = jnp.concatenate([p, feats[:, n * FE:(n + 1) * FE]], axis=1)
        pre = jnp.dot(x.astype(jnp.bfloat16), w0,
                      preferred_element_type=jnp.float32)
        psel = pre[:, 0:D_H]
        for i in range(1, E):
            psel = jnp.where(idn == i, pre[:, i * D_H:(i + 1) * D_H], psel)
        h = jnp.tanh(psel)
        o8 = jnp.dot(h.astype(jnp.bfloat16), w1,
                     preferred_element_type=jnp.float32)
        o = o8[:, 0:D_OUT]
        bsel = b1[0:1]
        for i in range(1, E):
            o = jnp.where(idn == i, o8[:, i * D_OUT:(i + 1) * D_OUT], o)
            bsel = jnp.where(idn == i, b1[i:i + 1], bsel)
        p = o + bsel
    out_ref[...] = jnp.maximum(p, 0.0)


def kernel(mod_feat_seq, p_in, W0, b0, W1, b1, mod_id_seq):
    w0cat = jnp.transpose(W0, (2, 0, 1)).reshape(D_IN, E * D_H)
    w0full = jnp.concatenate(
        [w0cat[:D_OUT],
         w0cat[D_OUT:],
         b0.reshape(1, E * D_H),
         jnp.zeros((FE - FEAT - 1, E * D_H), b0.dtype)],
        axis=0).astype(jnp.bfloat16)
    w1all = jnp.transpose(W1, (2, 0, 1)).reshape(D_H, E * D_OUT)
    w1all = w1all.astype(jnp.bfloat16)

    pad = jnp.zeros((B, L, FE - FEAT), mod_feat_seq.dtype)
    pad = pad.at[:, :, 0].set(1.0)
    feats = jnp.concatenate([mod_feat_seq, pad], axis=2).reshape(B, L * FE)
    ids = mod_id_seq.astype(jnp.int32)

    BB = 1024
    grid = (B // BB,)
    return pl.pallas_call(
        _fused_kernel,
        grid=grid,
        in_specs=[
            pl.BlockSpec((BB, L * FE), lambda b: (b, 0)),
            pl.BlockSpec((BB, D_OUT), lambda b: (b, 0)),
            pl.BlockSpec((XW, E * D_H), lambda b: (0, 0)),
            pl.BlockSpec((D_H, E * D_OUT), lambda b: (0, 0)),
            pl.BlockSpec((E, D_OUT), lambda b: (0, 0)),
            pl.BlockSpec((BB, L), lambda b: (b, 0)),
        ],
        out_specs=pl.BlockSpec((BB, D_OUT), lambda b: (b, 0)),
        out_shape=jax.ShapeDtypeStruct((B, D_OUT), jnp.float32),
        compiler_params=pltpu.CompilerParams(
            dimension_semantics=("parallel",)),
    )(feats, p_in, w0full, w1all, b1, ids)
